# initial kernel scaffold (unmeasured)
import jax
import jax.numpy as jnp
from jax import lax
from jax.experimental import pallas as pl
from jax.experimental.pallas import tpu as pltpu


def kernel(
    x,
):
    def body(*refs):
        pass

    out_shape = jax.ShapeDtypeStruct(..., jnp.float32)
    return pl.pallas_call(body, out_shape=out_shape)(...)


# baseline (device time: 846598 ns/iter reference)
import jax
import jax.numpy as jnp
from jax import lax
from jax.experimental import pallas as pl
from jax.experimental.pallas import tpu as pltpu

M = 16384
N = 1024
HALF = M // 2
C = 16
R = HALF // C
S = 4


def kernel(x):
    assert x.shape == (M, N), x.shape

    def body(x_ref, out_ref,
             xsend, local, xrecv, red, yrecv,
             in_xsend_sems, in_local_sems,
             x_send_sems, x_recv_sems,
             y_send_sems, y_recv_sems,
             out_red_sems, out_y_sems,
             credit_x, credit_y):
        my_x = lax.axis_index("x")
        my_y = lax.axis_index("y")
        my_z = lax.axis_index("z")
        xp = (1 - my_x, my_y, my_z)
        y_par = lax.rem(my_y, 2)
        yp = (my_x, my_y + 1 - 2 * y_par, my_z)

        r = lax.rem(my_x + my_y, 2)
        keep_off = r * HALF
        send_off = (1 - r) * HALF

        def stage_in(k):
            d_xs = pltpu.make_async_copy(
                x_ref.at[pl.ds(send_off + k * R, R), :],
                xsend.at[k % S],
                in_xsend_sems.at[k],
            )
            d_lo = pltpu.make_async_copy(
                x_ref.at[pl.ds(keep_off + k * R, R), :],
                local.at[k % S],
                in_local_sems.at[k],
            )
            d_xs.start()
            d_lo.start()
            return d_xs, d_lo

        bar = pltpu.get_barrier_semaphore()
        for nbr in (xp, yp):
            pl.semaphore_signal(bar, inc=1, device_id=nbr,
                                device_id_type=pl.DeviceIdType.MESH)
        pl.semaphore_wait(bar, 2)

        in_dmas = {}
        for k in range(S):
            in_dmas[k] = stage_in(k)

        x_rdmas = {}
        y_rdmas = {}
        out_red_dmas = {}
        out_y_dmas = {}

        for k in range(C):
            sl = k % S

            in_dmas[k][0].wait()
            if k >= S:
                pl.semaphore_wait(credit_x, 1)
            x_rdmas[k] = pltpu.make_async_remote_copy(
                src_ref=xsend.at[sl],
                dst_ref=xrecv.at[sl],
                send_sem=x_send_sems.at[k],
                recv_sem=x_recv_sems.at[k],
                device_id=xp,
                device_id_type=pl.DeviceIdType.MESH,
            )
            x_rdmas[k].start()

            in_dmas[k][1].wait()
            x_rdmas[k].wait_recv()
            if k >= S:
                y_rdmas[k - S].wait_send()
                out_red_dmas[k - S].wait()
            red[sl] = local[sl] + xrecv[sl]
            if k <= C - S - 1:
                pl.semaphore_signal(credit_x, inc=1, device_id=xp,
                                    device_id_type=pl.DeviceIdType.MESH)

            if k >= S:
                pl.semaphore_wait(credit_y, 1)
            y_rdmas[k] = pltpu.make_async_remote_copy(
                src_ref=red.at[sl],
                dst_ref=yrecv.at[sl],
                send_sem=y_send_sems.at[k],
                recv_sem=y_recv_sems.at[k],
                device_id=yp,
                device_id_type=pl.DeviceIdType.MESH,
            )
            y_rdmas[k].start()
            out_red_dmas[k] = pltpu.make_async_copy(
                red.at[sl],
                out_ref.at[pl.ds(keep_off + k * R, R), :],
                out_red_sems.at[k],
            )
            out_red_dmas[k].start()

            y_rdmas[k].wait_recv()
            out_y_dmas[k] = pltpu.make_async_copy(
                yrecv.at[sl],
                out_ref.at[pl.ds(send_off + k * R, R), :],
                out_y_sems.at[k],
            )
            out_y_dmas[k].start()
            out_y_dmas[k].wait()
            if k <= C - S - 1:
                pl.semaphore_signal(credit_y, inc=1, device_id=yp,
                                    device_id_type=pl.DeviceIdType.MESH)

            if k + S < C:
                x_rdmas[k].wait_send()
                in_dmas[k + S] = stage_in(k + S)

        for k in range(C - S, C):
            x_rdmas[k].wait_send()
            y_rdmas[k].wait_send()
            out_red_dmas[k].wait()

    vmem = pltpu.MemorySpace.VMEM
    return pl.pallas_call(
        body,
        out_shape=jax.ShapeDtypeStruct((M, N), x.dtype),
        in_specs=[pl.BlockSpec(memory_space=pl.ANY)],
        out_specs=pl.BlockSpec(memory_space=pl.ANY),
        scratch_shapes=[
            pltpu.VMEM((S, R, N), x.dtype),
            pltpu.VMEM((S, R, N), x.dtype),
            pltpu.VMEM((S, R, N), x.dtype),
            pltpu.VMEM((S, R, N), x.dtype),
            pltpu.VMEM((S, R, N), x.dtype),
            pltpu.SemaphoreType.DMA((C,)),
            pltpu.SemaphoreType.DMA((C,)),
            pltpu.SemaphoreType.DMA((C,)),
            pltpu.SemaphoreType.DMA((C,)),
            pltpu.SemaphoreType.DMA((C,)),
            pltpu.SemaphoreType.DMA((C,)),
            pltpu.SemaphoreType.DMA((C,)),
            pltpu.SemaphoreType.DMA((C,)),
            pltpu.SemaphoreType.REGULAR,
            pltpu.SemaphoreType.REGULAR,
        ],
        compiler_params=pltpu.CompilerParams(
            collective_id=0,
            vmem_limit_bytes=96 * 1024 * 1024,
        ),
    )(x)


# device time: 490951 ns/iter; 1.7244x vs baseline; 1.7244x over previous
import jax
import jax.numpy as jnp
from jax import lax
from jax.experimental import pallas as pl
from jax.experimental.pallas import tpu as pltpu

M = 16384
N = 1024
HALF = M // 2
C = 16
R = HALF // C
S = 4
DLAG = 2


def kernel(x):
    assert x.shape == (M, N), x.shape

    def body(x_ref, out_ref,
             xsend, local, xrecv, red, yrecv,
             in_xsend_sems, in_local_sems,
             x_send_sems, x_recv_sems,
             y_send_sems, y_recv_sems,
             out_red_sems, out_y_sems,
             credit_x, credit_y):
        my_x = lax.axis_index("x")
        my_y = lax.axis_index("y")
        my_z = lax.axis_index("z")
        xp = (1 - my_x, my_y, my_z)
        y_par = lax.rem(my_y, 2)
        yp = (my_x, my_y + 1 - 2 * y_par, my_z)

        r = lax.rem(my_x + my_y, 2)
        keep_off = r * HALF
        send_off = (1 - r) * HALF

        def stage_in(k):
            d_xs = pltpu.make_async_copy(
                x_ref.at[pl.ds(send_off + k * R, R), :],
                xsend.at[k % S],
                in_xsend_sems.at[k],
            )
            d_lo = pltpu.make_async_copy(
                x_ref.at[pl.ds(keep_off + k * R, R), :],
                local.at[k % S],
                in_local_sems.at[k],
            )
            d_xs.start()
            d_lo.start()
            return d_xs, d_lo

        bar = pltpu.get_barrier_semaphore()
        for nbr in (xp, yp):
            pl.semaphore_signal(bar, inc=1, device_id=nbr,
                                device_id_type=pl.DeviceIdType.MESH)
        pl.semaphore_wait(bar, 2)

        in_dmas = {}
        for k in range(S):
            in_dmas[k] = stage_in(k)

        x_rdmas = {}
        y_rdmas = {}
        out_red_dmas = {}
        out_y_dmas = {}

        for k in range(C):
            sl = k % S

            in_dmas[k][0].wait()
            if k >= S:
                pl.semaphore_wait(credit_x, 1)
            x_rdmas[k] = pltpu.make_async_remote_copy(
                src_ref=xsend.at[sl],
                dst_ref=xrecv.at[sl],
                send_sem=x_send_sems.at[k],
                recv_sem=x_recv_sems.at[k],
                device_id=xp,
                device_id_type=pl.DeviceIdType.MESH,
            )
            x_rdmas[k].start()

            in_dmas[k][1].wait()
            x_rdmas[k].wait_recv()
            if k >= S:
                y_rdmas[k - S].wait_send()
                out_red_dmas[k - S].wait()
            red[sl] = local[sl] + xrecv[sl]
            if k <= C - S - 1:
                pl.semaphore_signal(credit_x, inc=1, device_id=xp,
                                    device_id_type=pl.DeviceIdType.MESH)

            if k >= S:
                pl.semaphore_wait(credit_y, 1)
            y_rdmas[k] = pltpu.make_async_remote_copy(
                src_ref=red.at[sl],
                dst_ref=yrecv.at[sl],
                send_sem=y_send_sems.at[k],
                recv_sem=y_recv_sems.at[k],
                device_id=yp,
                device_id_type=pl.DeviceIdType.MESH,
            )
            y_rdmas[k].start()
            out_red_dmas[k] = pltpu.make_async_copy(
                red.at[sl],
                out_ref.at[pl.ds(keep_off + k * R, R), :],
                out_red_sems.at[k],
            )
            out_red_dmas[k].start()

            if k >= DLAG:
                j = k - DLAG
                jl = j % S
                y_rdmas[j].wait_recv()
                out_y_dmas[j] = pltpu.make_async_copy(
                    yrecv.at[jl],
                    out_ref.at[pl.ds(send_off + j * R, R), :],
                    out_y_sems.at[j],
                )
                out_y_dmas[j].start()
                out_y_dmas[j].wait()
                if j <= C - S - 1:
                    pl.semaphore_signal(credit_y, inc=1, device_id=yp,
                                        device_id_type=pl.DeviceIdType.MESH)

            if k + S < C:
                x_rdmas[k].wait_send()
                in_dmas[k + S] = stage_in(k + S)

        for j in range(C - DLAG, C):
            jl = j % S
            y_rdmas[j].wait_recv()
            out_y_dmas[j] = pltpu.make_async_copy(
                yrecv.at[jl],
                out_ref.at[pl.ds(send_off + j * R, R), :],
                out_y_sems.at[j],
            )
            out_y_dmas[j].start()
            out_y_dmas[j].wait()
            if j <= C - S - 1:
                pl.semaphore_signal(credit_y, inc=1, device_id=yp,
                                    device_id_type=pl.DeviceIdType.MESH)
        for k in range(C - S, C):
            x_rdmas[k].wait_send()
            y_rdmas[k].wait_send()
            out_red_dmas[k].wait()

    vmem = pltpu.MemorySpace.VMEM
    return pl.pallas_call(
        body,
        out_shape=jax.ShapeDtypeStruct((M, N), x.dtype),
        in_specs=[pl.BlockSpec(memory_space=pl.ANY)],
        out_specs=pl.BlockSpec(memory_space=pl.ANY),
        scratch_shapes=[
            pltpu.VMEM((S, R, N), x.dtype),
            pltpu.VMEM((S, R, N), x.dtype),
            pltpu.VMEM((S, R, N), x.dtype),
            pltpu.VMEM((S, R, N), x.dtype),
            pltpu.VMEM((S, R, N), x.dtype),
            pltpu.SemaphoreType.DMA((C,)),
            pltpu.SemaphoreType.DMA((C,)),
            pltpu.SemaphoreType.DMA((C,)),
            pltpu.SemaphoreType.DMA((C,)),
            pltpu.SemaphoreType.DMA((C,)),
            pltpu.SemaphoreType.DMA((C,)),
            pltpu.SemaphoreType.DMA((C,)),
            pltpu.SemaphoreType.DMA((C,)),
            pltpu.SemaphoreType.REGULAR,
            pltpu.SemaphoreType.REGULAR,
        ],
        compiler_params=pltpu.CompilerParams(
            collective_id=0,
            vmem_limit_bytes=96 * 1024 * 1024,
        ),
    )(x)


# device time: 436505 ns/iter; 1.9395x vs baseline; 1.1247x over previous
import jax
import jax.numpy as jnp
from jax import lax
from jax.experimental import pallas as pl
from jax.experimental.pallas import tpu as pltpu

M = 16384
N = 1024
HALF = M // 2
C = 16
R = HALF // C
S = 4
DLAG = 2


def kernel(x):
    assert x.shape == (M, N), x.shape

    def body(x_ref, out_ref,
             xsend, local, xrecv, red, yrecv,
             in_xsend_sems, in_local_sems,
             x_send_sems, x_recv_sems,
             y_send_sems, y_recv_sems,
             out_red_sems, out_y_sems,
             credit_x, credit_y):
        my_x = lax.axis_index("x")
        my_y = lax.axis_index("y")
        my_z = lax.axis_index("z")
        xp = (1 - my_x, my_y, my_z)
        y_par = lax.rem(my_y, 2)
        yp = (my_x, my_y + 1 - 2 * y_par, my_z)

        r = lax.rem(my_x + my_y, 2)
        keep_off = r * HALF
        send_off = (1 - r) * HALF

        def stage_in(k):
            d_xs = pltpu.make_async_copy(
                x_ref.at[pl.ds(send_off + k * R, R), :],
                xsend.at[k % S],
                in_xsend_sems.at[k],
            )
            d_lo = pltpu.make_async_copy(
                x_ref.at[pl.ds(keep_off + k * R, R), :],
                local.at[k % S],
                in_local_sems.at[k],
            )
            d_xs.start()
            d_lo.start()
            return d_xs, d_lo

        bar = pltpu.get_barrier_semaphore()
        for nbr in (xp, yp):
            pl.semaphore_signal(bar, inc=1, device_id=nbr,
                                device_id_type=pl.DeviceIdType.MESH)
        pl.semaphore_wait(bar, 2)

        in_dmas = {}
        for k in range(S):
            in_dmas[k] = stage_in(k)

        x_rdmas = {}
        y_rdmas = {}
        out_red_dmas = {}
        out_y_dmas = {}

        def start_x(k):
            in_dmas[k][0].wait()
            if k >= S:
                pl.semaphore_wait(credit_x, 1)
            x_rdmas[k] = pltpu.make_async_remote_copy(
                src_ref=xsend.at[k % S],
                dst_ref=xrecv.at[k % S],
                send_sem=x_send_sems.at[k],
                recv_sem=x_recv_sems.at[k],
                device_id=xp,
                device_id_type=pl.DeviceIdType.MESH,
            )
            x_rdmas[k].start()

        start_x(0)
        for k in range(C):
            sl = k % S

            if k + 1 < C:
                start_x(k + 1)

            in_dmas[k][1].wait()
            x_rdmas[k].wait_recv()
            if k >= S:
                y_rdmas[k - S].wait_send()
                out_red_dmas[k - S].wait()
            red[sl] = local[sl] + xrecv[sl]
            if k <= C - S - 1:
                pl.semaphore_signal(credit_x, inc=1, device_id=xp,
                                    device_id_type=pl.DeviceIdType.MESH)

            if k >= S:
                pl.semaphore_wait(credit_y, 1)
            y_rdmas[k] = pltpu.make_async_remote_copy(
                src_ref=red.at[sl],
                dst_ref=yrecv.at[sl],
                send_sem=y_send_sems.at[k],
                recv_sem=y_recv_sems.at[k],
                device_id=yp,
                device_id_type=pl.DeviceIdType.MESH,
            )
            y_rdmas[k].start()
            out_red_dmas[k] = pltpu.make_async_copy(
                red.at[sl],
                out_ref.at[pl.ds(keep_off + k * R, R), :],
                out_red_sems.at[k],
            )
            out_red_dmas[k].start()

            if k >= DLAG:
                j = k - DLAG
                jl = j % S
                y_rdmas[j].wait_recv()
                out_y_dmas[j] = pltpu.make_async_copy(
                    yrecv.at[jl],
                    out_ref.at[pl.ds(send_off + j * R, R), :],
                    out_y_sems.at[j],
                )
                out_y_dmas[j].start()
                out_y_dmas[j].wait()
                if j <= C - S - 1:
                    pl.semaphore_signal(credit_y, inc=1, device_id=yp,
                                        device_id_type=pl.DeviceIdType.MESH)

            if k + S < C:
                x_rdmas[k].wait_send()
                in_dmas[k + S] = stage_in(k + S)

        for j in range(C - DLAG, C):
            jl = j % S
            y_rdmas[j].wait_recv()
            out_y_dmas[j] = pltpu.make_async_copy(
                yrecv.at[jl],
                out_ref.at[pl.ds(send_off + j * R, R), :],
                out_y_sems.at[j],
            )
            out_y_dmas[j].start()
            out_y_dmas[j].wait()
            if j <= C - S - 1:
                pl.semaphore_signal(credit_y, inc=1, device_id=yp,
                                    device_id_type=pl.DeviceIdType.MESH)
        for k in range(C - S, C):
            x_rdmas[k].wait_send()
            y_rdmas[k].wait_send()
            out_red_dmas[k].wait()

    vmem = pltpu.MemorySpace.VMEM
    return pl.pallas_call(
        body,
        out_shape=jax.ShapeDtypeStruct((M, N), x.dtype),
        in_specs=[pl.BlockSpec(memory_space=pl.ANY)],
        out_specs=pl.BlockSpec(memory_space=pl.ANY),
        scratch_shapes=[
            pltpu.VMEM((S, R, N), x.dtype),
            pltpu.VMEM((S, R, N), x.dtype),
            pltpu.VMEM((S, R, N), x.dtype),
            pltpu.VMEM((S, R, N), x.dtype),
            pltpu.VMEM((S, R, N), x.dtype),
            pltpu.SemaphoreType.DMA((C,)),
            pltpu.SemaphoreType.DMA((C,)),
            pltpu.SemaphoreType.DMA((C,)),
            pltpu.SemaphoreType.DMA((C,)),
            pltpu.SemaphoreType.DMA((C,)),
            pltpu.SemaphoreType.DMA((C,)),
            pltpu.SemaphoreType.DMA((C,)),
            pltpu.SemaphoreType.DMA((C,)),
            pltpu.SemaphoreType.REGULAR,
            pltpu.SemaphoreType.REGULAR,
        ],
        compiler_params=pltpu.CompilerParams(
            collective_id=0,
            vmem_limit_bytes=96 * 1024 * 1024,
        ),
    )(x)
